# baseline (speedup 1.0000x reference)
import jax
import jax.numpy as jnp
import numpy as np
from jax.experimental import pallas as pl
from jax.experimental.pallas import tpu as pltpu

_EPS = 0.01
_FEAT = 256
_K = 16
_LOG2PI = float(np.log(2.0 * np.pi))


def _gmm_body(obs_ref, eps_ref, u_ref, W1_ref, b1_ref, W2_ref, b2_ref,
              W3w_ref, W3mu_ref, W3sig_ref, b3w_ref, b3mu_ref, b3sig_ref,
              seg_ref, act_ref, ent_ref, mean_ref):
    f32 = jnp.float32
    h = jnp.maximum(
        jnp.dot(obs_ref[...], W1_ref[...], preferred_element_type=f32)
        + b1_ref[...], 0.0)
    h = jnp.maximum(
        jnp.dot(h, W2_ref[...], preferred_element_type=f32) + b2_ref[...],
        0.0)
    logw = jnp.dot(h, W3w_ref[...], preferred_element_type=f32) + b3w_ref[...]
    hb = h.astype(jnp.bfloat16)
    mu = (jnp.dot(hb, W3mu_ref[...], preferred_element_type=f32)
          + b3mu_ref[...]).astype(jnp.bfloat16)
    lsig = jnp.clip(
        jnp.dot(hb, W3sig_ref[...], preferred_element_type=f32)
        + b3sig_ref[...], -5.0, 2.0).astype(jnp.bfloat16)

    m = jnp.max(logw, axis=-1, keepdims=True)
    lse = m + jnp.log(jnp.sum(jnp.exp(logw - m), axis=-1, keepdims=True))
    log_ws = logw - lse

    gumbel = -jnp.log(-jnp.log(u_ref[...]))
    scores = log_ws + gumbel
    best = jnp.max(scores, axis=-1, keepdims=True)

    bt = obs_ref.shape[0]
    bf16 = jnp.bfloat16
    mu_z = jnp.zeros((bt, _FEAT), bf16)
    lsig_z = jnp.zeros((bt, _FEAT), bf16)
    found = jnp.zeros((bt, 1), dtype=jnp.bool_)
    for k in range(_K):
        sel = (scores[:, k:k + 1] >= best) & (~found)
        found = found | sel
        mu_z = jnp.where(sel, mu[:, k * _FEAT:(k + 1) * _FEAT], mu_z)
        lsig_z = jnp.where(sel, lsig[:, k * _FEAT:(k + 1) * _FEAT], lsig_z)

    x = mu_z.astype(f32) + jnp.exp(lsig_z.astype(f32)) * eps_ref[...]

    xt = jnp.concatenate([x] * _K, axis=-1)
    lsf = lsig.astype(f32)
    d = (xt - mu.astype(f32)) * jnp.exp(-lsf)
    S = -0.5 * d * d - lsf
    s = jnp.dot(S, seg_ref[...], preferred_element_type=f32)
    log_p_k = log_ws + s - 0.5 * _FEAT * _LOG2PI
    mk = jnp.max(log_p_k, axis=-1, keepdims=True)
    log_p_x = mk + jnp.log(
        jnp.sum(jnp.exp(log_p_k - mk), axis=-1, keepdims=True))

    mean = jnp.zeros((bt, _FEAT), f32)
    for k in range(_K):
        mean = mean + (jnp.exp(log_ws[:, k:k + 1])
                       * mu[:, k * _FEAT:(k + 1) * _FEAT].astype(f32))

    act = jnp.tanh(x)
    t2 = jnp.tanh(act)
    squash = jnp.sum(jnp.log(1.0 - t2 * t2 + _EPS), axis=-1, keepdims=True)
    act_ref[...] = act
    ent_ref[...] = -(log_p_x - squash)
    mean_ref[...] = jnp.tanh(mean)


def _run(obs, eps, u, W1, b1, W2, b2, W3, b3):
    B, OBS = obs.shape
    H1 = W1.shape[1]
    H2 = W2.shape[1]
    KF = _K * _FEAT

    W3r = W3.reshape(H2, _K, 2 * _FEAT + 1)
    W3w = W3r[:, :, 0]
    W3mu = W3r[:, :, 1:1 + _FEAT].reshape(H2, KF).astype(jnp.bfloat16)
    W3sig = W3r[:, :, 1 + _FEAT:].reshape(H2, KF).astype(jnp.bfloat16)
    b3r = b3.reshape(_K, 2 * _FEAT + 1)
    b3w = b3r[:, 0].reshape(1, _K)
    b3mu = b3r[:, 1:1 + _FEAT].reshape(1, KF)
    b3sig = b3r[:, 1 + _FEAT:].reshape(1, KF)
    b1r = b1.reshape(1, H1)
    b2r = b2.reshape(1, H2)
    seg = jnp.asarray(np.kron(np.eye(_K, dtype=np.float32),
                              np.ones((_FEAT, 1), dtype=np.float32)))

    BT = 256
    grid = (B // BT,)

    def row(i):
        return (i, 0)

    def rep(i):
        return (0, 0)

    act, ent, mean = pl.pallas_call(
        _gmm_body,
        grid=grid,
        in_specs=[
            pl.BlockSpec((BT, OBS), row),
            pl.BlockSpec((BT, _FEAT), row),
            pl.BlockSpec((BT, _K), row),
            pl.BlockSpec((OBS, H1), rep),
            pl.BlockSpec((1, H1), rep),
            pl.BlockSpec((H1, H2), rep),
            pl.BlockSpec((1, H2), rep),
            pl.BlockSpec((H2, _K), rep),
            pl.BlockSpec((H2, KF), rep),
            pl.BlockSpec((H2, KF), rep),
            pl.BlockSpec((1, _K), rep),
            pl.BlockSpec((1, KF), rep),
            pl.BlockSpec((1, KF), rep),
            pl.BlockSpec((KF, _K), rep),
        ],
        out_specs=[
            pl.BlockSpec((BT, _FEAT), row),
            pl.BlockSpec((BT, 1), row),
            pl.BlockSpec((BT, _FEAT), row),
        ],
        out_shape=[
            jax.ShapeDtypeStruct((B, _FEAT), jnp.float32),
            jax.ShapeDtypeStruct((B, 1), jnp.float32),
            jax.ShapeDtypeStruct((B, _FEAT), jnp.float32),
        ],
        compiler_params=pltpu.CompilerParams(
            dimension_semantics=("parallel",)),
    )(obs, eps, u, W1, b1r, W2, b2r, W3w, W3mu, W3sig, b3w, b3mu, b3sig,
      seg)
    return act, ent, mean


def kernel(obs, eps, u, W1, b1, W2, b2, W3, b3):
    return _run(obs, eps, u, W1, b1, W2, b2, W3, b3)


# speedup vs baseline: 1.0202x; 1.0202x over previous
import jax
import jax.numpy as jnp
import numpy as np
from jax.experimental import pallas as pl
from jax.experimental.pallas import tpu as pltpu

_EPS = 0.01
_FEAT = 256
_K = 16
_LOG2PI = float(np.log(2.0 * np.pi))


def _gmm_body(obs_ref, eps_ref, u_ref, W1_ref, b1_ref, W2_ref, b2_ref,
              W3w_ref, W3mu_ref, W3sig_ref, b3w_ref, b3mu_ref, b3sig_ref,
              seg_ref, act_ref, ent_ref, mean_ref):
    f32 = jnp.float32
    h = jnp.maximum(
        jnp.dot(obs_ref[...], W1_ref[...], preferred_element_type=f32)
        + b1_ref[...], 0.0)
    h = jnp.maximum(
        jnp.dot(h, W2_ref[...], preferred_element_type=f32) + b2_ref[...],
        0.0)
    logw = jnp.dot(h, W3w_ref[...], preferred_element_type=f32) + b3w_ref[...]
    hb = h.astype(jnp.bfloat16)
    mu = (jnp.dot(hb, W3mu_ref[...], preferred_element_type=f32)
          + b3mu_ref[...]).astype(jnp.bfloat16)
    lsig = jnp.clip(
        jnp.dot(hb, W3sig_ref[...], preferred_element_type=f32)
        + b3sig_ref[...], -5.0, 2.0).astype(jnp.bfloat16)

    m = jnp.max(logw, axis=-1, keepdims=True)
    lse = m + jnp.log(jnp.sum(jnp.exp(logw - m), axis=-1, keepdims=True))
    log_ws = logw - lse

    gumbel = -jnp.log(-jnp.log(u_ref[...]))
    scores = log_ws + gumbel
    best = jnp.max(scores, axis=-1, keepdims=True)

    bt = obs_ref.shape[0]
    bf16 = jnp.bfloat16
    mu_z = jnp.zeros((bt, _FEAT), bf16)
    lsig_z = jnp.zeros((bt, _FEAT), bf16)
    for k in reversed(range(_K)):
        sel = scores[:, k:k + 1] >= best
        mu_z = jnp.where(sel, mu[:, k * _FEAT:(k + 1) * _FEAT], mu_z)
        lsig_z = jnp.where(sel, lsig[:, k * _FEAT:(k + 1) * _FEAT], lsig_z)

    x = mu_z.astype(f32) + jnp.exp(lsig_z.astype(f32)) * eps_ref[...]

    xb = x.astype(bf16)
    xt = jnp.concatenate([xb] * _K, axis=-1)
    d = (xt - mu) * jnp.exp(-lsig)
    S = -0.5 * d * d - lsig
    s = jnp.dot(S, seg_ref[...], preferred_element_type=f32)
    log_p_k = log_ws + s - 0.5 * _FEAT * _LOG2PI
    mk = jnp.max(log_p_k, axis=-1, keepdims=True)
    log_p_x = mk + jnp.log(
        jnp.sum(jnp.exp(log_p_k - mk), axis=-1, keepdims=True))

    mean = jnp.zeros((bt, _FEAT), f32)
    for k in range(_K):
        mean = mean + (jnp.exp(log_ws[:, k:k + 1])
                       * mu[:, k * _FEAT:(k + 1) * _FEAT].astype(f32))

    act = jnp.tanh(x)
    t2 = jnp.tanh(act)
    squash = jnp.sum(jnp.log(1.0 - t2 * t2 + _EPS), axis=-1, keepdims=True)
    act_ref[...] = act
    ent_ref[...] = -(log_p_x - squash)
    mean_ref[...] = jnp.tanh(mean)


def _run(obs, eps, u, W1, b1, W2, b2, W3, b3):
    B, OBS = obs.shape
    H1 = W1.shape[1]
    H2 = W2.shape[1]
    KF = _K * _FEAT

    W3r = W3.reshape(H2, _K, 2 * _FEAT + 1)
    W3w = W3r[:, :, 0]
    W3mu = W3r[:, :, 1:1 + _FEAT].reshape(H2, KF).astype(jnp.bfloat16)
    W3sig = W3r[:, :, 1 + _FEAT:].reshape(H2, KF).astype(jnp.bfloat16)
    b3r = b3.reshape(_K, 2 * _FEAT + 1)
    b3w = b3r[:, 0].reshape(1, _K)
    b3mu = b3r[:, 1:1 + _FEAT].reshape(1, KF)
    b3sig = b3r[:, 1 + _FEAT:].reshape(1, KF)
    b1r = b1.reshape(1, H1)
    b2r = b2.reshape(1, H2)
    seg = jnp.asarray(np.kron(np.eye(_K, dtype=np.float32),
                              np.ones((_FEAT, 1), dtype=np.float32)),
                      dtype=jnp.bfloat16)

    BT = 256
    grid = (B // BT,)

    def row(i):
        return (i, 0)

    def rep(i):
        return (0, 0)

    act, ent, mean = pl.pallas_call(
        _gmm_body,
        grid=grid,
        in_specs=[
            pl.BlockSpec((BT, OBS), row),
            pl.BlockSpec((BT, _FEAT), row),
            pl.BlockSpec((BT, _K), row),
            pl.BlockSpec((OBS, H1), rep),
            pl.BlockSpec((1, H1), rep),
            pl.BlockSpec((H1, H2), rep),
            pl.BlockSpec((1, H2), rep),
            pl.BlockSpec((H2, _K), rep),
            pl.BlockSpec((H2, KF), rep),
            pl.BlockSpec((H2, KF), rep),
            pl.BlockSpec((1, _K), rep),
            pl.BlockSpec((1, KF), rep),
            pl.BlockSpec((1, KF), rep),
            pl.BlockSpec((KF, _K), rep),
        ],
        out_specs=[
            pl.BlockSpec((BT, _FEAT), row),
            pl.BlockSpec((BT, 1), row),
            pl.BlockSpec((BT, _FEAT), row),
        ],
        out_shape=[
            jax.ShapeDtypeStruct((B, _FEAT), jnp.float32),
            jax.ShapeDtypeStruct((B, 1), jnp.float32),
            jax.ShapeDtypeStruct((B, _FEAT), jnp.float32),
        ],
        compiler_params=pltpu.CompilerParams(
            dimension_semantics=("parallel",)),
    )(obs, eps, u, W1, b1r, W2, b2r, W3w, W3mu, W3sig, b3w, b3mu, b3sig,
      seg)
    return act, ent, mean


def kernel(obs, eps, u, W1, b1, W2, b2, W3, b3):
    return _run(obs, eps, u, W1, b1, W2, b2, W3, b3)
